# baseline (speedup 1.0000x reference)
import jax
import jax.numpy as jnp
from jax.experimental import pallas as pl
from jax.experimental.pallas import tpu as pltpu

_B = 4096
_D = 128
_N = 2 * _B
_TEMP = 0.1
_BR = 2048
_NBLK = _N // _BR
_NPAIRS = _NBLK * (_NBLK + 1) // 2
_BASES = [I * _NBLK - I * (I - 1) // 2 for I in range(_NBLK)]

_LOG2E = 1.4426950408889634
_C = (_LOG2E / _TEMP) ** 0.5
_LN2 = 0.6931471805599453


def _ntxent_kernel(zi_ref, zj_ref, out_ref, rn_ref, acc_ref):
    k = pl.program_id(0)

    @pl.when(k == 0)
    def _init():
        r = jnp.concatenate([zi_ref[...], zj_ref[...]], axis=0)
        nrm = jnp.maximum(jnp.sqrt(jnp.sum(r * r, axis=1, keepdims=True)), 1e-12)
        rn_ref[...] = r * (_C / nrm)
        acc_ref[...] = jnp.zeros_like(acc_ref)

    i_blk = jnp.int32(0)
    base = jnp.int32(0)
    for t in range(1, _NBLK):
        hit = k >= _BASES[t]
        i_blk = jnp.where(hit, t, i_blk)
        base = jnp.where(hit, _BASES[t], base)
    j_blk = k - base + i_blk

    ri = rn_ref[pl.ds(i_blk * _BR, _BR), :]
    rj = rn_ref[pl.ds(j_blk * _BR, _BR), :]
    s2 = jax.lax.dot_general(
        ri, rj, (((1,), (1,)), ((), ())),
        preferred_element_type=jnp.float32,
    )
    e = jnp.exp2(s2.astype(jnp.bfloat16))
    rowsum = jnp.sum(e, axis=1, keepdims=True, dtype=jnp.float32)
    acc_ref[pl.ds(i_blk * _BR, _BR), :] += rowsum

    @pl.when(i_blk != j_blk)
    def _colsum():
        colsum = jnp.sum(e, axis=0, keepdims=True, dtype=jnp.float32)
        acc_ref[pl.ds(j_blk * _BR, _BR), :] += jnp.transpose(colsum, (1, 0))

    @pl.when(k == _NPAIRS - 1)
    def _epilogue():
        rn = rn_ref[...]
        selfdot = jnp.sum(rn * rn, axis=1, keepdims=True)
        tot = acc_ref[...] - jnp.exp2(selfdot)
        lse_sum = jnp.sum(jnp.log(tot))
        pos_sum = jnp.sum(rn_ref[0:_B, :] * rn_ref[_B:_N, :])
        out_ref[0, 0] = (lse_sum - 2.0 * _LN2 * pos_sum) * (1.0 / _N)


def kernel(z_i, z_j):
    out = pl.pallas_call(
        _ntxent_kernel,
        grid=(_NPAIRS,),
        in_specs=[
            pl.BlockSpec((_B, _D), lambda k: (0, 0)),
            pl.BlockSpec((_B, _D), lambda k: (0, 0)),
        ],
        out_specs=pl.BlockSpec(memory_space=pltpu.SMEM),
        out_shape=jax.ShapeDtypeStruct((1, 1), jnp.float32),
        scratch_shapes=[
            pltpu.VMEM((_N, _D), jnp.float32),
            pltpu.VMEM((_N, 1), jnp.float32),
        ],
    )(z_i, z_j)
    return out[0, 0]


# speedup vs baseline: 1.0489x; 1.0489x over previous
import jax
import jax.numpy as jnp
from jax.experimental import pallas as pl
from jax.experimental.pallas import tpu as pltpu

_B = 4096
_D = 128
_N = 2 * _B
_TEMP = 0.1
_BR = 2048
_NBLK = _N // _BR
_NPAIRS = _NBLK * (_NBLK + 1) // 2
_BASES = [I * _NBLK - I * (I - 1) // 2 for I in range(_NBLK)]

_LOG2E = 1.4426950408889634
_C = (_LOG2E / _TEMP) ** 0.5
_LN2 = 0.6931471805599453


def _ntxent_kernel(zi_ref, zj_ref, out_ref, rn_ref, acc_ref):
    k = pl.program_id(0)

    @pl.when(k == 0)
    def _init():
        r = jnp.concatenate([zi_ref[...], zj_ref[...]], axis=0)
        nrm = jnp.maximum(jnp.sqrt(jnp.sum(r * r, axis=1, keepdims=True)), 1e-12)
        rn_ref[...] = r * (_C / nrm)
        acc_ref[...] = jnp.zeros_like(acc_ref)

    for kk in (2 * k, 2 * k + 1):
        i_blk = jnp.int32(0)
        base = jnp.int32(0)
        for t in range(1, _NBLK):
            hit = kk >= _BASES[t]
            i_blk = jnp.where(hit, t, i_blk)
            base = jnp.where(hit, _BASES[t], base)
        j_blk = kk - base + i_blk

        ri = rn_ref[pl.ds(i_blk * _BR, _BR), :]
        rj = rn_ref[pl.ds(j_blk * _BR, _BR), :]
        s2 = jax.lax.dot_general(
            ri, rj, (((1,), (1,)), ((), ())),
            preferred_element_type=jnp.float32,
        )
        e = jnp.exp2(s2)
        rowsum = jnp.sum(e, axis=1, keepdims=True)
        acc_ref[pl.ds(i_blk * _BR, _BR), :] += rowsum

        @pl.when(i_blk != j_blk)
        def _colsum(e=e, j_blk=j_blk):
            colsum = jnp.sum(e, axis=0, keepdims=True)
            acc_ref[pl.ds(j_blk * _BR, _BR), :] += jnp.transpose(colsum, (1, 0))

    @pl.when(k == _NPAIRS // 2 - 1)
    def _epilogue():
        rn = rn_ref[...]
        selfdot = jnp.sum(rn * rn, axis=1, keepdims=True)
        tot = acc_ref[...] - jnp.exp2(selfdot)
        lse_sum = jnp.sum(jnp.log(tot))
        pos_sum = jnp.sum(rn_ref[0:_B, :] * rn_ref[_B:_N, :])
        out_ref[0, 0] = (lse_sum - 2.0 * _LN2 * pos_sum) * (1.0 / _N)


def kernel(z_i, z_j):
    out = pl.pallas_call(
        _ntxent_kernel,
        grid=(_NPAIRS // 2,),
        in_specs=[
            pl.BlockSpec((_B, _D), lambda k: (0, 0)),
            pl.BlockSpec((_B, _D), lambda k: (0, 0)),
        ],
        out_specs=pl.BlockSpec(memory_space=pltpu.SMEM),
        out_shape=jax.ShapeDtypeStruct((1, 1), jnp.float32),
        scratch_shapes=[
            pltpu.VMEM((_N, _D), jnp.float32),
            pltpu.VMEM((_N, 1), jnp.float32),
        ],
    )(z_i, z_j)
    return out[0, 0]


# speedup vs baseline: 1.1095x; 1.0577x over previous
import jax
import jax.numpy as jnp
from jax.experimental import pallas as pl
from jax.experimental.pallas import tpu as pltpu

_B = 4096
_D = 128
_N = 2 * _B
_TEMP = 0.1
_BR = 2048
_NBLK = _N // _BR
_NPAIRS = _NBLK * (_NBLK + 1) // 2
_BASES = [I * _NBLK - I * (I - 1) // 2 for I in range(_NBLK)]

_LOG2E = 1.4426950408889634
_C = (_LOG2E / _TEMP) ** 0.5
_LN2 = 0.6931471805599453


def _ntxent_kernel(zi_ref, zj_ref, out_ref, rn_ref, acc_ref):
    k = pl.program_id(0)

    @pl.when(k == 0)
    def _init():
        r = jnp.concatenate([zi_ref[...], zj_ref[...]], axis=0)
        nrm = jnp.maximum(jnp.sqrt(jnp.sum(r * r, axis=1, keepdims=True)), 1e-12)
        rn_ref[...] = r * (_C / nrm)
        acc_ref[...] = jnp.zeros_like(acc_ref)

    i_blk = jnp.int32(0)
    base = jnp.int32(0)
    for t in range(1, _NBLK):
        hit = k >= _BASES[t]
        i_blk = jnp.where(hit, t, i_blk)
        base = jnp.where(hit, _BASES[t], base)
    j_blk = k - base + i_blk

    rj = rn_ref[pl.ds(j_blk * _BR, _BR), :]
    cs = jnp.zeros((1, _BR), dtype=jnp.float32)
    for r in range(_BR // 128):
        rstrip = rn_ref[pl.ds(i_blk * _BR + r * 128, 128), :]
        s2r = jax.lax.dot_general(
            rstrip, rj, (((1,), (1,)), ((), ())),
            preferred_element_type=jnp.float32,
        )
        er = jnp.exp2(s2r)
        acc_ref[pl.ds(i_blk * _BR + r * 128, 128), :] += jnp.sum(
            er, axis=1, keepdims=True)
        cs = cs + jnp.sum(er, axis=0, keepdims=True)

    @pl.when(i_blk != j_blk)
    def _colsum():
        acc_ref[pl.ds(j_blk * _BR, _BR), :] += jnp.transpose(cs, (1, 0))

    @pl.when(k == _NPAIRS - 1)
    def _epilogue():
        rn = rn_ref[...]
        selfdot = jnp.sum(rn * rn, axis=1, keepdims=True)
        tot = acc_ref[...] - jnp.exp2(selfdot)
        lse_sum = jnp.sum(jnp.log(tot))
        pos_sum = jnp.sum(rn_ref[0:_B, :] * rn_ref[_B:_N, :])
        out_ref[0, 0] = (lse_sum - 2.0 * _LN2 * pos_sum) * (1.0 / _N)


def kernel(z_i, z_j):
    out = pl.pallas_call(
        _ntxent_kernel,
        grid=(_NPAIRS,),
        in_specs=[
            pl.BlockSpec((_B, _D), lambda k: (0, 0)),
            pl.BlockSpec((_B, _D), lambda k: (0, 0)),
        ],
        out_specs=pl.BlockSpec(memory_space=pltpu.SMEM),
        out_shape=jax.ShapeDtypeStruct((1, 1), jnp.float32),
        scratch_shapes=[
            pltpu.VMEM((_N, _D), jnp.float32),
            pltpu.VMEM((_N, 1), jnp.float32),
        ],
    )(z_i, z_j)
    return out[0, 0]


# speedup vs baseline: 1.1322x; 1.0205x over previous
import jax
import jax.numpy as jnp
from jax.experimental import pallas as pl
from jax.experimental.pallas import tpu as pltpu

_B = 4096
_D = 128
_N = 2 * _B
_TEMP = 0.1
_BR = 2048
_NBLK = _N // _BR
_NPAIRS = _NBLK * (_NBLK + 1) // 2
_BASES = [I * _NBLK - I * (I - 1) // 2 for I in range(_NBLK)]

_LOG2E = 1.4426950408889634
_C = (_LOG2E / _TEMP) ** 0.5
_LN2 = 0.6931471805599453


def _ntxent_kernel(zi_ref, zj_ref, out_ref, rn_ref, acc_ref):
    k = pl.program_id(0)

    @pl.when(k == 0)
    def _init():
        r = jnp.concatenate([zi_ref[...], zj_ref[...]], axis=0)
        nrm = jnp.maximum(jnp.sqrt(jnp.sum(r * r, axis=1, keepdims=True)), 1e-12)
        rn_ref[...] = r * (_C / nrm)
        acc_ref[...] = jnp.zeros_like(acc_ref)

    i_blk = jnp.int32(0)
    base = jnp.int32(0)
    for t in range(1, _NBLK):
        hit = k >= _BASES[t]
        i_blk = jnp.where(hit, t, i_blk)
        base = jnp.where(hit, _BASES[t], base)
    j_blk = k - base + i_blk

    rj = rn_ref[pl.ds(j_blk * _BR, _BR), :]
    cs = jnp.zeros((1, _BR), dtype=jnp.float32)
    for r in range(_BR // 256):
        rstrip = rn_ref[pl.ds(i_blk * _BR + r * 256, 256), :]
        s2r = jax.lax.dot_general(
            rstrip, rj, (((1,), (1,)), ((), ())),
            preferred_element_type=jnp.float32,
        )
        er = jnp.exp2(s2r)
        acc_ref[pl.ds(i_blk * _BR + r * 256, 256), :] += jnp.sum(
            er, axis=1, keepdims=True)
        cs = cs + jnp.sum(er, axis=0, keepdims=True)

    @pl.when(i_blk != j_blk)
    def _colsum():
        acc_ref[pl.ds(j_blk * _BR, _BR), :] += jnp.transpose(cs, (1, 0))

    @pl.when(k == _NPAIRS - 1)
    def _epilogue():
        rn = rn_ref[...]
        selfdot = jnp.sum(rn * rn, axis=1, keepdims=True)
        tot = acc_ref[...] - jnp.exp2(selfdot)
        lse_sum = jnp.sum(jnp.log(tot))
        pos_sum = jnp.sum(rn_ref[0:_B, :] * rn_ref[_B:_N, :])
        out_ref[0, 0] = (lse_sum - 2.0 * _LN2 * pos_sum) * (1.0 / _N)


def kernel(z_i, z_j):
    out = pl.pallas_call(
        _ntxent_kernel,
        grid=(_NPAIRS,),
        in_specs=[
            pl.BlockSpec((_B, _D), lambda k: (0, 0)),
            pl.BlockSpec((_B, _D), lambda k: (0, 0)),
        ],
        out_specs=pl.BlockSpec(memory_space=pltpu.SMEM),
        out_shape=jax.ShapeDtypeStruct((1, 1), jnp.float32),
        scratch_shapes=[
            pltpu.VMEM((_N, _D), jnp.float32),
            pltpu.VMEM((_N, 1), jnp.float32),
        ],
    )(z_i, z_j)
    return out[0, 0]


# speedup vs baseline: 1.1537x; 1.0190x over previous
import jax
import jax.numpy as jnp
from jax.experimental import pallas as pl
from jax.experimental.pallas import tpu as pltpu

_B = 4096
_D = 128
_N = 2 * _B
_TEMP = 0.1
_BR = 2048
_NBLK = _N // _BR
_NPAIRS = _NBLK * (_NBLK + 1) // 2
_BASES = [I * _NBLK - I * (I - 1) // 2 for I in range(_NBLK)]

_LOG2E = 1.4426950408889634
_C = (_LOG2E / _TEMP) ** 0.5
_LN2 = 0.6931471805599453


def _ntxent_kernel(zi_ref, zj_ref, out_ref, rn_ref, acc_ref):
    k = pl.program_id(0)

    @pl.when(k == 0)
    def _init():
        r = jnp.concatenate([zi_ref[...], zj_ref[...]], axis=0)
        nrm = jnp.maximum(jnp.sqrt(jnp.sum(r * r, axis=1, keepdims=True)), 1e-12)
        rn_ref[...] = r * (_C / nrm)
        acc_ref[...] = jnp.zeros_like(acc_ref)

    i_blk = jnp.int32(0)
    base = jnp.int32(0)
    for t in range(1, _NBLK):
        hit = k >= _BASES[t]
        i_blk = jnp.where(hit, t, i_blk)
        base = jnp.where(hit, _BASES[t], base)
    j_blk = k - base + i_blk

    rj = rn_ref[pl.ds(j_blk * _BR, _BR), :]
    cs = jnp.zeros((1, _BR), dtype=jnp.float32)
    for r in range(_BR // 512):
        rstrip = rn_ref[pl.ds(i_blk * _BR + r * 512, 512), :]
        s2r = jax.lax.dot_general(
            rstrip, rj, (((1,), (1,)), ((), ())),
            preferred_element_type=jnp.float32,
        )
        er = jnp.exp2(s2r)
        acc_ref[pl.ds(i_blk * _BR + r * 512, 512), :] += jnp.sum(
            er, axis=1, keepdims=True)
        cs = cs + jnp.sum(er, axis=0, keepdims=True)

    @pl.when(i_blk != j_blk)
    def _colsum():
        acc_ref[pl.ds(j_blk * _BR, _BR), :] += jnp.transpose(cs, (1, 0))

    @pl.when(k == _NPAIRS - 1)
    def _epilogue():
        rn = rn_ref[...]
        selfdot = jnp.sum(rn * rn, axis=1, keepdims=True)
        tot = acc_ref[...] - jnp.exp2(selfdot)
        lse_sum = jnp.sum(jnp.log(tot))
        pos_sum = jnp.sum(rn_ref[0:_B, :] * rn_ref[_B:_N, :])
        out_ref[0, 0] = (lse_sum - 2.0 * _LN2 * pos_sum) * (1.0 / _N)


def kernel(z_i, z_j):
    out = pl.pallas_call(
        _ntxent_kernel,
        grid=(_NPAIRS,),
        in_specs=[
            pl.BlockSpec((_B, _D), lambda k: (0, 0)),
            pl.BlockSpec((_B, _D), lambda k: (0, 0)),
        ],
        out_specs=pl.BlockSpec(memory_space=pltpu.SMEM),
        out_shape=jax.ShapeDtypeStruct((1, 1), jnp.float32),
        scratch_shapes=[
            pltpu.VMEM((_N, _D), jnp.float32),
            pltpu.VMEM((_N, 1), jnp.float32),
        ],
    )(z_i, z_j)
    return out[0, 0]


# speedup vs baseline: 1.1540x; 1.0002x over previous
import jax
import jax.numpy as jnp
from jax.experimental import pallas as pl
from jax.experimental.pallas import tpu as pltpu

_B = 4096
_D = 128
_N = 2 * _B
_TEMP = 0.1
_BR = 2048
_NBLK = _N // _BR
_NPAIRS = _NBLK * (_NBLK + 1) // 2
_BASES = [I * _NBLK - I * (I - 1) // 2 for I in range(_NBLK)]

_LOG2E = 1.4426950408889634
_C = (_LOG2E / _TEMP) ** 0.5
_LN2 = 0.6931471805599453


def _ntxent_kernel(zi_ref, zj_ref, out_ref, rn_ref, acc_ref):
    k = pl.program_id(0)

    @pl.when(k == 0)
    def _init():
        r = jnp.concatenate([zi_ref[...], zj_ref[...]], axis=0)
        nrm = jnp.maximum(jnp.sqrt(jnp.sum(r * r, axis=1, keepdims=True)), 1e-12)
        rn_ref[...] = r * (_C / nrm)
        acc_ref[...] = jnp.zeros_like(acc_ref)

    i_blk = jnp.int32(0)
    base = jnp.int32(0)
    for t in range(1, _NBLK):
        hit = k >= _BASES[t]
        i_blk = jnp.where(hit, t, i_blk)
        base = jnp.where(hit, _BASES[t], base)
    j_blk = k - base + i_blk

    rj = rn_ref[pl.ds(j_blk * _BR, _BR), :]
    cs = jnp.zeros((1, _BR), dtype=jnp.float32)
    for r in range(_BR // 1024):
        rstrip = rn_ref[pl.ds(i_blk * _BR + r * 1024, 1024), :]
        s2r = jax.lax.dot_general(
            rstrip, rj, (((1,), (1,)), ((), ())),
            preferred_element_type=jnp.float32,
        )
        er = jnp.exp2(s2r)
        acc_ref[pl.ds(i_blk * _BR + r * 1024, 1024), :] += jnp.sum(
            er, axis=1, keepdims=True)
        cs = cs + jnp.sum(er, axis=0, keepdims=True)

    @pl.when(i_blk != j_blk)
    def _colsum():
        acc_ref[pl.ds(j_blk * _BR, _BR), :] += jnp.transpose(cs, (1, 0))

    @pl.when(k == _NPAIRS - 1)
    def _epilogue():
        rn = rn_ref[...]
        selfdot = jnp.sum(rn * rn, axis=1, keepdims=True)
        tot = acc_ref[...] - jnp.exp2(selfdot)
        lse_sum = jnp.sum(jnp.log(tot))
        pos_sum = jnp.sum(rn_ref[0:_B, :] * rn_ref[_B:_N, :])
        out_ref[0, 0] = (lse_sum - 2.0 * _LN2 * pos_sum) * (1.0 / _N)


def kernel(z_i, z_j):
    out = pl.pallas_call(
        _ntxent_kernel,
        grid=(_NPAIRS,),
        in_specs=[
            pl.BlockSpec((_B, _D), lambda k: (0, 0)),
            pl.BlockSpec((_B, _D), lambda k: (0, 0)),
        ],
        out_specs=pl.BlockSpec(memory_space=pltpu.SMEM),
        out_shape=jax.ShapeDtypeStruct((1, 1), jnp.float32),
        scratch_shapes=[
            pltpu.VMEM((_N, _D), jnp.float32),
            pltpu.VMEM((_N, 1), jnp.float32),
        ],
    )(z_i, z_j)
    return out[0, 0]


# speedup vs baseline: 1.2777x; 1.1072x over previous
import jax
import jax.numpy as jnp
from jax.experimental import pallas as pl
from jax.experimental.pallas import tpu as pltpu

_B = 4096
_D = 128
_N = 2 * _B
_TEMP = 0.1
_BR = 2048
_NBLK = _N // _BR
_NPAIRS = _NBLK * (_NBLK + 1) // 2
_BASES = [I * _NBLK - I * (I - 1) // 2 for I in range(_NBLK)]
_ST = 512
_NST = _BR // _ST

_LOG2E = 1.4426950408889634
_C = (_LOG2E / _TEMP) ** 0.5
_LN2 = 0.6931471805599453


def _ntxent_kernel(zi_ref, zj_ref, out_ref, rn_ref, acc_ref):
    k = pl.program_id(0)

    @pl.when(k == 0)
    def _init():
        r = jnp.concatenate([zi_ref[...], zj_ref[...]], axis=0)
        nrm = jnp.maximum(jnp.sqrt(jnp.sum(r * r, axis=1, keepdims=True)), 1e-12)
        rn_ref[...] = r * (_C / nrm)
        acc_ref[...] = jnp.zeros_like(acc_ref)

    i_blk = jnp.int32(0)
    base = jnp.int32(0)
    for t in range(1, _NBLK):
        hit = k >= _BASES[t]
        i_blk = jnp.where(hit, t, i_blk)
        base = jnp.where(hit, _BASES[t], base)
    j_blk = k - base + i_blk

    @pl.when(i_blk != j_blk)
    def _offdiag():
        rj = rn_ref[pl.ds(j_blk * _BR, _BR), :]
        cs = jnp.zeros((1, _BR), dtype=jnp.float32)
        for r in range(_NST):
            rstrip = rn_ref[pl.ds(i_blk * _BR + r * _ST, _ST), :]
            s2r = jax.lax.dot_general(
                rstrip, rj, (((1,), (1,)), ((), ())),
                preferred_element_type=jnp.float32,
            )
            er = jnp.exp2(s2r)
            acc_ref[pl.ds(i_blk * _BR + r * _ST, _ST), :] += jnp.sum(
                er, axis=1, keepdims=True)
            cs = cs + jnp.sum(er, axis=0, keepdims=True)
        acc_ref[pl.ds(j_blk * _BR, _BR), :] += jnp.transpose(cs, (1, 0))

    @pl.when(i_blk == j_blk)
    def _diag():
        b0 = i_blk * _BR
        segs = [jnp.zeros((1, _ST), dtype=jnp.float32) for _ in range(_NST)]
        for r in range(_NST):
            w = _BR - r * _ST
            rstrip = rn_ref[pl.ds(b0 + r * _ST, _ST), :]
            rjc = rn_ref[pl.ds(b0 + r * _ST, w), :]
            s2r = jax.lax.dot_general(
                rstrip, rjc, (((1,), (1,)), ((), ())),
                preferred_element_type=jnp.float32,
            )
            er = jnp.exp2(s2r)
            acc_ref[pl.ds(b0 + r * _ST, _ST), :] += jnp.sum(
                er, axis=1, keepdims=True)
            if r < _NST - 1:
                partial = jnp.sum(er[:, _ST:], axis=0, keepdims=True)
                for m in range(r + 1, _NST):
                    segs[m] = segs[m] + partial[:, (m - r - 1) * _ST:(m - r) * _ST]
        cs = jnp.concatenate(segs, axis=1)
        acc_ref[pl.ds(b0, _BR), :] += jnp.transpose(cs, (1, 0))

    @pl.when(k == _NPAIRS - 1)
    def _epilogue():
        rn = rn_ref[...]
        selfdot = jnp.sum(rn * rn, axis=1, keepdims=True)
        tot = acc_ref[...] - jnp.exp2(selfdot)
        lse_sum = jnp.sum(jnp.log(tot))
        pos_sum = jnp.sum(rn_ref[0:_B, :] * rn_ref[_B:_N, :])
        out_ref[0, 0] = (lse_sum - 2.0 * _LN2 * pos_sum) * (1.0 / _N)


def kernel(z_i, z_j):
    out = pl.pallas_call(
        _ntxent_kernel,
        grid=(_NPAIRS,),
        in_specs=[
            pl.BlockSpec((_B, _D), lambda k: (0, 0)),
            pl.BlockSpec((_B, _D), lambda k: (0, 0)),
        ],
        out_specs=pl.BlockSpec(memory_space=pltpu.SMEM),
        out_shape=jax.ShapeDtypeStruct((1, 1), jnp.float32),
        scratch_shapes=[
            pltpu.VMEM((_N, _D), jnp.float32),
            pltpu.VMEM((_N, 1), jnp.float32),
        ],
    )(z_i, z_j)
    return out[0, 0]


# speedup vs baseline: 1.3983x; 1.0944x over previous
import jax
import jax.numpy as jnp
from jax.experimental import pallas as pl
from jax.experimental.pallas import tpu as pltpu

_B = 4096
_D = 128
_N = 2 * _B
_TEMP = 0.1
_BR = 2048
_NBLK = _N // _BR
_ST = 512
_NST = _BR // _ST

_LOG2E = 1.4426950408889634
_C = (_LOG2E / _TEMP) ** 0.5
_LN2 = 0.6931471805599453


def _ntxent_kernel(zi_ref, zj_ref, out_ref, rn_ref, acc_ref):
    r = jnp.concatenate([zi_ref[...], zj_ref[...]], axis=0)
    nrm = jnp.maximum(jnp.sqrt(jnp.sum(r * r, axis=1, keepdims=True)), 1e-12)
    rn_ref[...] = r * (_C / nrm)
    acc_ref[...] = jnp.zeros_like(acc_ref)

    for i_blk in range(_NBLK):
        for j_blk in range(i_blk, _NBLK):
            if i_blk != j_blk:
                rj = rn_ref[j_blk * _BR:(j_blk + 1) * _BR, :]
                cs = jnp.zeros((1, _BR), dtype=jnp.float32)
                for s in range(_NST):
                    lo = i_blk * _BR + s * _ST
                    rstrip = rn_ref[lo:lo + _ST, :]
                    s2r = jax.lax.dot_general(
                        rstrip, rj, (((1,), (1,)), ((), ())),
                        preferred_element_type=jnp.float32,
                    )
                    er = jnp.exp2(s2r)
                    acc_ref[lo:lo + _ST, :] += jnp.sum(er, axis=1, keepdims=True)
                    cs = cs + jnp.sum(er, axis=0, keepdims=True)
                jlo = j_blk * _BR
                acc_ref[jlo:jlo + _BR, :] += jnp.transpose(cs, (1, 0))
            else:
                b0 = i_blk * _BR
                segs = [jnp.zeros((1, _ST), dtype=jnp.float32)
                        for _ in range(_NST)]
                for s in range(_NST):
                    w = _BR - s * _ST
                    lo = b0 + s * _ST
                    rstrip = rn_ref[lo:lo + _ST, :]
                    rjc = rn_ref[lo:lo + w, :]
                    s2r = jax.lax.dot_general(
                        rstrip, rjc, (((1,), (1,)), ((), ())),
                        preferred_element_type=jnp.float32,
                    )
                    er = jnp.exp2(s2r)
                    acc_ref[lo:lo + _ST, :] += jnp.sum(er, axis=1, keepdims=True)
                    if s < _NST - 1:
                        partial = jnp.sum(er[:, _ST:], axis=0, keepdims=True)
                        for m in range(s + 1, _NST):
                            segs[m] = segs[m] + partial[
                                :, (m - s - 1) * _ST:(m - s) * _ST]
                cs = jnp.concatenate(segs, axis=1)
                acc_ref[b0:b0 + _BR, :] += jnp.transpose(cs, (1, 0))

    rn = rn_ref[...]
    selfdot = jnp.sum(rn * rn, axis=1, keepdims=True)
    tot = acc_ref[...] - jnp.exp2(selfdot)
    lse_sum = jnp.sum(jnp.log(tot))
    pos_sum = jnp.sum(rn_ref[0:_B, :] * rn_ref[_B:_N, :])
    out_ref[0, 0] = (lse_sum - 2.0 * _LN2 * pos_sum) * (1.0 / _N)


def kernel(z_i, z_j):
    out = pl.pallas_call(
        _ntxent_kernel,
        out_specs=pl.BlockSpec(memory_space=pltpu.SMEM),
        out_shape=jax.ShapeDtypeStruct((1, 1), jnp.float32),
        scratch_shapes=[
            pltpu.VMEM((_N, _D), jnp.float32),
            pltpu.VMEM((_N, 1), jnp.float32),
        ],
    )(z_i, z_j)
    return out[0, 0]


# speedup vs baseline: 1.4165x; 1.0130x over previous
import jax
import jax.numpy as jnp
from jax.experimental import pallas as pl
from jax.experimental.pallas import tpu as pltpu

_B = 4096
_D = 128
_N = 2 * _B
_TEMP = 0.1
_BR = 2048
_NBLK = _N // _BR
_ST = 256
_NST = _BR // _ST

_LOG2E = 1.4426950408889634
_C = (_LOG2E / _TEMP) ** 0.5
_LN2 = 0.6931471805599453


def _ntxent_kernel(zi_ref, zj_ref, out_ref, rn_ref, acc_ref):
    r = jnp.concatenate([zi_ref[...], zj_ref[...]], axis=0)
    nrm = jnp.maximum(jnp.sqrt(jnp.sum(r * r, axis=1, keepdims=True)), 1e-12)
    rn_ref[...] = r * (_C / nrm)
    acc_ref[...] = jnp.zeros_like(acc_ref)

    for i_blk in range(_NBLK):
        for j_blk in range(i_blk, _NBLK):
            if i_blk != j_blk:
                rj = rn_ref[j_blk * _BR:(j_blk + 1) * _BR, :]
                cs = jnp.zeros((1, _BR), dtype=jnp.float32)
                for s in range(_NST):
                    lo = i_blk * _BR + s * _ST
                    rstrip = rn_ref[lo:lo + _ST, :]
                    s2r = jax.lax.dot_general(
                        rstrip, rj, (((1,), (1,)), ((), ())),
                        preferred_element_type=jnp.float32,
                    )
                    er = jnp.exp2(s2r)
                    acc_ref[lo:lo + _ST, :] += jnp.sum(er, axis=1, keepdims=True)
                    cs = cs + jnp.sum(er, axis=0, keepdims=True)
                jlo = j_blk * _BR
                acc_ref[jlo:jlo + _BR, :] += jnp.transpose(cs, (1, 0))
            else:
                b0 = i_blk * _BR
                segs = [jnp.zeros((1, _ST), dtype=jnp.float32)
                        for _ in range(_NST)]
                for s in range(_NST):
                    w = _BR - s * _ST
                    lo = b0 + s * _ST
                    rstrip = rn_ref[lo:lo + _ST, :]
                    rjc = rn_ref[lo:lo + w, :]
                    s2r = jax.lax.dot_general(
                        rstrip, rjc, (((1,), (1,)), ((), ())),
                        preferred_element_type=jnp.float32,
                    )
                    er = jnp.exp2(s2r)
                    acc_ref[lo:lo + _ST, :] += jnp.sum(er, axis=1, keepdims=True)
                    if s < _NST - 1:
                        partial = jnp.sum(er[:, _ST:], axis=0, keepdims=True)
                        for m in range(s + 1, _NST):
                            segs[m] = segs[m] + partial[
                                :, (m - s - 1) * _ST:(m - s) * _ST]
                cs = jnp.concatenate(segs, axis=1)
                acc_ref[b0:b0 + _BR, :] += jnp.transpose(cs, (1, 0))

    rn = rn_ref[...]
    selfdot = jnp.sum(rn * rn, axis=1, keepdims=True)
    tot = acc_ref[...] - jnp.exp2(selfdot)
    lse_sum = jnp.sum(jnp.log(tot))
    pos_sum = jnp.sum(rn_ref[0:_B, :] * rn_ref[_B:_N, :])
    out_ref[0, 0] = (lse_sum - 2.0 * _LN2 * pos_sum) * (1.0 / _N)


def kernel(z_i, z_j):
    out = pl.pallas_call(
        _ntxent_kernel,
        out_specs=pl.BlockSpec(memory_space=pltpu.SMEM),
        out_shape=jax.ShapeDtypeStruct((1, 1), jnp.float32),
        scratch_shapes=[
            pltpu.VMEM((_N, _D), jnp.float32),
            pltpu.VMEM((_N, 1), jnp.float32),
        ],
    )(z_i, z_j)
    return out[0, 0]
